# baseline (speedup 1.0000x reference)
import functools

import jax
import jax.numpy as jnp
from jax.experimental import pallas as pl
from jax.experimental.pallas import tpu as pltpu

_EPS = 1e-8
_NEG = -3.0e38
_K = 5
_FOLD_W = 256


def _extract5(vals, labs, width):
    col = jax.lax.broadcasted_iota(jnp.int32, vals.shape, 1)
    out_v, out_l = [], []
    w = vals
    for _ in range(_K):
        m = jnp.max(w, axis=1, keepdims=True)
        cand = jnp.where(w == m, col, width)
        amin = jnp.min(cand, axis=1, keepdims=True)
        sel = col == amin
        lab_t = jnp.sum(jnp.where(sel, labs, 0.0), axis=1, keepdims=True)
        out_v.append(m)
        out_l.append(lab_t)
        w = jnp.where(sel, _NEG, w)
    return out_v, out_l


def _knn_body(nblk, eblk, q_ref, e_ref, lab_ref, k_ref, out_ref,
              qn_ref, rv_ref, rl_ref, ws_ref, cv_ref, cl_ref, ll_ref,
              coll_ref):
    i = pl.program_id(0)
    nq = q_ref.shape[0]
    pad = jnp.full((nq, 3), _NEG, jnp.float32)
    zpad = jnp.zeros((nq, 3), jnp.float32)

    @pl.when(i == 0)
    def _init():
        q = q_ref[...]
        qn = q / jnp.maximum(
            jnp.sqrt(jnp.sum(q * q, axis=1, keepdims=True)), _EPS)
        qn_ref[...] = qn
        rv_ref[...] = jnp.full(rv_ref.shape, _NEG, jnp.float32)
        rl_ref[...] = jnp.zeros(rl_ref.shape, jnp.float32)

    @pl.when(i < nblk)
    def _produce():
        slot = jax.lax.rem(i, 2)
        e = e_ref[...]
        en = e / jnp.maximum(
            jnp.sqrt(jnp.sum(e * e, axis=1, keepdims=True)), _EPS)
        sims = jax.lax.dot_general(
            qn_ref[...], en, (((1,), (1,)), ((), ())),
            preferred_element_type=jnp.float32)

        tau = rv_ref[:, _K - 1:_K]
        gt = sims > tau
        ws = jnp.where(gt, sims, _NEG)
        ws_ref[slot] = ws
        labrow = jnp.broadcast_to(lab_ref[0, 0, :][None, :], (nq, eblk))
        ll_ref[slot] = labrow

        nch = eblk // _FOLD_W
        sv = [ws[:, c * _FOLD_W:(c + 1) * _FOLD_W] for c in range(nch)]
        sl = [labrow[:, c * _FOLD_W:(c + 1) * _FOLD_W] for c in range(nch)]
        fc = [gt[:, c * _FOLD_W:(c + 1) * _FOLD_W].astype(jnp.float32)
              for c in range(nch)]
        fv1, fv2, fl1, fl2, nc = [], [], [], [], []
        for a in range(0, nch, 2):
            b = a + 1
            m = sv[b] > sv[a]
            fv1.append(jnp.where(m, sv[b], sv[a]))
            fl1.append(jnp.where(m, sl[b], sl[a]))
            fv2.append(jnp.where(m, sv[a], sv[b]))
            fl2.append(jnp.where(m, sl[a], sl[b]))
            nc.append(fc[a] + fc[b])
        fc = nc
        while len(fv1) > 1:
            nv1, nv2, nl1, nl2, nc = [], [], [], [], []
            for a in range(0, len(fv1), 2):
                b = a + 1
                m1 = fv1[b] > fv1[a]
                lv = jnp.where(m1, fv1[a], fv1[b])
                ll = jnp.where(m1, fl1[a], fl1[b])
                s2 = jnp.where(m1, fv2[b], fv2[a])
                s2l = jnp.where(m1, fl2[b], fl2[a])
                m2 = s2 > lv
                nv1.append(jnp.where(m1, fv1[b], fv1[a]))
                nl1.append(jnp.where(m1, fl1[b], fl1[a]))
                nv2.append(jnp.where(m2, s2, lv))
                nl2.append(jnp.where(m2, s2l, ll))
                nc.append(fc[a] + fc[b])
            fv1, fv2, fl1, fl2, fc = nv1, nv2, nl1, nl2, nc

        cv_ref[slot] = jnp.concatenate([fv1[0], fv2[0]], axis=1)
        cl_ref[slot] = jnp.concatenate([fl1[0], fl2[0]], axis=1)
        coll_ref[slot] = jnp.max(fc[0])

    @pl.when(i > 0)
    def _consume():
        pslot = jax.lax.rem(i - 1, 2)
        collision = coll_ref[pslot] > 2.5

        @pl.when(collision)
        def _fallback():
            wv5, wl5 = _extract5(ws_ref[pslot], ll_ref[pslot], eblk)
            fbv = jnp.concatenate(wv5 + [pad], axis=1)
            fbl = jnp.concatenate(wl5 + [zpad], axis=1)
            mv = jnp.concatenate([rv_ref[...], fbv], axis=1)
            ml = jnp.concatenate([rl_ref[...], fbl], axis=1)
            nv5, nl5 = _extract5(mv, ml, 16)
            rv_ref[...] = jnp.concatenate(nv5 + [pad], axis=1)
            rl_ref[...] = jnp.concatenate(nl5 + [zpad], axis=1)

        @pl.when(jnp.logical_not(collision))
        def _merge():
            bv5, bl5 = _extract5(cv_ref[pslot], cl_ref[pslot], 2 * _FOLD_W)
            bv = jnp.concatenate(bv5 + [pad], axis=1)
            bl = jnp.concatenate(bl5 + [zpad], axis=1)
            mv = jnp.concatenate([rv_ref[...], bv], axis=1)
            ml = jnp.concatenate([rl_ref[...], bl], axis=1)
            nv5, nl5 = _extract5(mv, ml, 16)
            rv_ref[...] = jnp.concatenate(nv5 + [pad], axis=1)
            rl_ref[...] = jnp.concatenate(nl5 + [zpad], axis=1)

    @pl.when(i == nblk)
    def _fin():
        lab_sum = jnp.sum(rl_ref[:, :_K], axis=1)
        out_ref[0, :] = jnp.round(lab_sum / k_ref[0, 0])


def kernel(queries, embeddings, labels, k):
    nq, d = queries.shape
    n, _ = embeddings.shape
    eblk = 4096
    nblk = n // eblk

    labs3 = labels.reshape(nblk, 1, eblk)
    k_arr = jnp.asarray(k, jnp.float32).reshape(1, 1)

    out = pl.pallas_call(
        functools.partial(_knn_body, nblk, eblk),
        grid=(nblk + 1,),
        in_specs=[
            pl.BlockSpec((nq, d), lambda i: (0, 0)),
            pl.BlockSpec((eblk, d), lambda i: (jnp.minimum(i, nblk - 1), 0)),
            pl.BlockSpec((1, 1, eblk),
                         lambda i: (jnp.minimum(i, nblk - 1), 0, 0)),
            pl.BlockSpec(memory_space=pltpu.SMEM),
        ],
        out_specs=pl.BlockSpec((1, nq), lambda i: (0, 0)),
        out_shape=jax.ShapeDtypeStruct((1, nq), jnp.float32),
        scratch_shapes=[
            pltpu.VMEM((nq, d), jnp.float32),
            pltpu.VMEM((nq, 8), jnp.float32),
            pltpu.VMEM((nq, 8), jnp.float32),
            pltpu.VMEM((2, nq, eblk), jnp.float32),
            pltpu.VMEM((2, nq, 2 * _FOLD_W), jnp.float32),
            pltpu.VMEM((2, nq, 2 * _FOLD_W), jnp.float32),
            pltpu.VMEM((2, nq, eblk), jnp.float32),
            pltpu.SMEM((2,), jnp.float32),
        ],
        compiler_params=pltpu.CompilerParams(
            dimension_semantics=("arbitrary",),
            vmem_limit_bytes=120 * 1024 * 1024,
        ),
    )(queries, embeddings, labs3, k_arr)
    return out.reshape(nq)


# speedup vs baseline: 1.0323x; 1.0323x over previous
import functools

import jax
import jax.numpy as jnp
from jax.experimental import pallas as pl
from jax.experimental.pallas import tpu as pltpu

_EPS = 1e-8
_NEG = -3.0e38
_K = 5
_FOLD_W = 256


def _extract5(vals, labs, width):
    col = jax.lax.broadcasted_iota(jnp.int32, vals.shape, 1)
    out_v, out_l = [], []
    w = vals
    for _ in range(_K):
        m = jnp.max(w, axis=1, keepdims=True)
        cand = jnp.where(w == m, col, width)
        amin = jnp.min(cand, axis=1, keepdims=True)
        sel = col == amin
        lab_t = jnp.sum(jnp.where(sel, labs, 0.0), axis=1, keepdims=True)
        out_v.append(m)
        out_l.append(lab_t)
        w = jnp.where(sel, _NEG, w)
    return out_v, out_l


def _knn_body(nblk, eblk, q_ref, e_ref, lab_ref, k_ref, out_ref,
              qn_ref, rv_ref, rl_ref, ws_ref, bv_ref, bl_ref):
    i = pl.program_id(0)
    nq = q_ref.shape[0]

    @pl.when(i == 0)
    def _init():
        q = q_ref[...]
        qn = q / jnp.maximum(
            jnp.sqrt(jnp.sum(q * q, axis=1, keepdims=True)), _EPS)
        qn_ref[...] = qn
        rv_ref[...] = jnp.full(rv_ref.shape, _NEG, jnp.float32)
        rl_ref[...] = jnp.zeros(rl_ref.shape, jnp.float32)

    e = e_ref[...]
    en = e / jnp.maximum(
        jnp.sqrt(jnp.sum(e * e, axis=1, keepdims=True)), _EPS)
    sims = jax.lax.dot_general(
        qn_ref[...], en, (((1,), (1,)), ((), ())),
        preferred_element_type=jnp.float32)

    tau = rv_ref[:, _K - 1:_K]
    gt = sims > tau
    ws = jnp.where(gt, sims, _NEG)
    ws_ref[...] = ws
    labrow = jnp.broadcast_to(lab_ref[0, 0, :][None, :], (nq, eblk))

    nch = eblk // _FOLD_W
    sv = [ws[:, c * _FOLD_W:(c + 1) * _FOLD_W] for c in range(nch)]
    sl = [labrow[:, c * _FOLD_W:(c + 1) * _FOLD_W] for c in range(nch)]
    fc = [gt[:, c * _FOLD_W:(c + 1) * _FOLD_W].astype(jnp.float32)
          for c in range(nch)]
    fv1, fv2, fl1, fl2, nc = [], [], [], [], []
    for a in range(0, nch, 2):
        b = a + 1
        m = sv[b] > sv[a]
        fv1.append(jnp.where(m, sv[b], sv[a]))
        fl1.append(jnp.where(m, sl[b], sl[a]))
        fv2.append(jnp.where(m, sv[a], sv[b]))
        fl2.append(jnp.where(m, sl[a], sl[b]))
        nc.append(fc[a] + fc[b])
    fc = nc
    while len(fv1) > 1:
        nv1, nv2, nl1, nl2, nc = [], [], [], [], []
        for a in range(0, len(fv1), 2):
            b = a + 1
            m1 = fv1[b] > fv1[a]
            lv = jnp.where(m1, fv1[a], fv1[b])
            ll = jnp.where(m1, fl1[a], fl1[b])
            s2 = jnp.where(m1, fv2[b], fv2[a])
            s2l = jnp.where(m1, fl2[b], fl2[a])
            m2 = s2 > lv
            nv1.append(jnp.where(m1, fv1[b], fv1[a]))
            nl1.append(jnp.where(m1, fl1[b], fl1[a]))
            nv2.append(jnp.where(m2, s2, lv))
            nl2.append(jnp.where(m2, s2l, ll))
            nc.append(fc[a] + fc[b])
        fv1, fv2, fl1, fl2, fc = nv1, nv2, nl1, nl2, nc

    cat_v = jnp.concatenate([fv1[0], fv2[0]], axis=1)
    cat_l = jnp.concatenate([fl1[0], fl2[0]], axis=1)
    bv5, bl5 = _extract5(cat_v, cat_l, 2 * _FOLD_W)
    pad = jnp.full((nq, 3), _NEG, jnp.float32)
    bv_ref[...] = jnp.concatenate(bv5 + [pad], axis=1)
    bl_ref[...] = jnp.concatenate(bl5 + [jnp.zeros((nq, 3))], axis=1)

    collision = jnp.max(fc[0]) > 2.5

    @pl.when(collision)
    def _fallback():
        wv5, wl5 = _extract5(ws_ref[...], labrow, eblk)
        bv_ref[...] = jnp.concatenate(wv5 + [pad], axis=1)
        bl_ref[...] = jnp.concatenate(wl5 + [jnp.zeros((nq, 3))], axis=1)

    mv = jnp.concatenate([rv_ref[...], bv_ref[...]], axis=1)
    ml = jnp.concatenate([rl_ref[...], bl_ref[...]], axis=1)
    nv5, nl5 = _extract5(mv, ml, 16)
    rv_ref[...] = jnp.concatenate(nv5 + [pad], axis=1)
    rl_ref[...] = jnp.concatenate(nl5 + [jnp.zeros((nq, 3))], axis=1)

    @pl.when(i == nblk - 1)
    def _fin():
        lab_sum = jnp.sum(rl_ref[:, :_K], axis=1)
        out_ref[0, :] = jnp.round(lab_sum / k_ref[0, 0])


def kernel(queries, embeddings, labels, k):
    nq, d = queries.shape
    n, _ = embeddings.shape
    eblk = 4096
    nblk = n // eblk

    labs3 = labels.reshape(nblk, 1, eblk)
    k_arr = jnp.asarray(k, jnp.float32).reshape(1, 1)

    out = pl.pallas_call(
        functools.partial(_knn_body, nblk, eblk),
        grid=(nblk,),
        in_specs=[
            pl.BlockSpec((nq, d), lambda i: (0, 0)),
            pl.BlockSpec((eblk, d), lambda i: (i, 0)),
            pl.BlockSpec((1, 1, eblk), lambda i: (i, 0, 0)),
            pl.BlockSpec(memory_space=pltpu.SMEM),
        ],
        out_specs=pl.BlockSpec((1, nq), lambda i: (0, 0)),
        out_shape=jax.ShapeDtypeStruct((1, nq), jnp.float32),
        scratch_shapes=[
            pltpu.VMEM((nq, d), jnp.float32),
            pltpu.VMEM((nq, 8), jnp.float32),
            pltpu.VMEM((nq, 8), jnp.float32),
            pltpu.VMEM((nq, eblk), jnp.float32),
            pltpu.VMEM((nq, 8), jnp.float32),
            pltpu.VMEM((nq, 8), jnp.float32),
        ],
        compiler_params=pltpu.CompilerParams(
            dimension_semantics=("arbitrary",),
            vmem_limit_bytes=120 * 1024 * 1024,
        ),
    )(queries, embeddings, labs3, k_arr)
    return out.reshape(nq)


# speedup vs baseline: 1.0522x; 1.0193x over previous
import functools

import jax
import jax.numpy as jnp
from jax.experimental import pallas as pl
from jax.experimental.pallas import tpu as pltpu

_EPS = 1e-8
_NEG = -3.0e38
_K = 5
_FOLD_W = 128


def _extract5(vals, labs, width):
    col = jax.lax.broadcasted_iota(jnp.int32, vals.shape, 1)
    out_v, out_l = [], []
    w = vals
    for _ in range(_K):
        m = jnp.max(w, axis=1, keepdims=True)
        cand = jnp.where(w == m, col, width)
        amin = jnp.min(cand, axis=1, keepdims=True)
        sel = col == amin
        lab_t = jnp.sum(jnp.where(sel, labs, 0.0), axis=1, keepdims=True)
        out_v.append(m)
        out_l.append(lab_t)
        w = jnp.where(sel, _NEG, w)
    return out_v, out_l


def _knn_body(nblk, eblk, q_ref, e_ref, lab_ref, k_ref, out_ref,
              qn_ref, rv_ref, rl_ref, ws_ref, bv_ref, bl_ref):
    i = pl.program_id(0)
    nq = q_ref.shape[0]

    @pl.when(i == 0)
    def _init():
        q = q_ref[...]
        qn = q / jnp.maximum(
            jnp.sqrt(jnp.sum(q * q, axis=1, keepdims=True)), _EPS)
        qn_ref[...] = qn
        rv_ref[...] = jnp.full(rv_ref.shape, _NEG, jnp.float32)
        rl_ref[...] = jnp.zeros(rl_ref.shape, jnp.float32)

    e = e_ref[...]
    en = e / jnp.maximum(
        jnp.sqrt(jnp.sum(e * e, axis=1, keepdims=True)), _EPS)
    sims = jax.lax.dot_general(
        qn_ref[...], en, (((1,), (1,)), ((), ())),
        preferred_element_type=jnp.float32)

    tau = rv_ref[:, _K - 1:_K]
    ws_ref[...] = sims
    labrow = jnp.broadcast_to(lab_ref[0, 0, :][None, :], (nq, eblk))

    nch = eblk // _FOLD_W
    gtc = [sims[:, c * _FOLD_W:(c + 1) * _FOLD_W] > tau for c in range(nch)]
    sv = [jnp.where(gtc[c], sims[:, c * _FOLD_W:(c + 1) * _FOLD_W], _NEG)
          for c in range(nch)]
    sl = [labrow[:, c * _FOLD_W:(c + 1) * _FOLD_W] for c in range(nch)]
    fc = [g.astype(jnp.float32) for g in gtc]
    fv1, fv2, fl1, fl2, nc = [], [], [], [], []
    for a in range(0, nch, 2):
        b = a + 1
        m = sv[b] > sv[a]
        fv1.append(jnp.where(m, sv[b], sv[a]))
        fl1.append(jnp.where(m, sl[b], sl[a]))
        fv2.append(jnp.where(m, sv[a], sv[b]))
        fl2.append(jnp.where(m, sl[a], sl[b]))
        nc.append(fc[a] + fc[b])
    fc = nc
    while len(fv1) > 1:
        nv1, nv2, nl1, nl2, nc = [], [], [], [], []
        for a in range(0, len(fv1), 2):
            b = a + 1
            m1 = fv1[b] > fv1[a]
            lv = jnp.where(m1, fv1[a], fv1[b])
            ll = jnp.where(m1, fl1[a], fl1[b])
            s2 = jnp.where(m1, fv2[b], fv2[a])
            s2l = jnp.where(m1, fl2[b], fl2[a])
            m2 = s2 > lv
            nv1.append(jnp.where(m1, fv1[b], fv1[a]))
            nl1.append(jnp.where(m1, fl1[b], fl1[a]))
            nv2.append(jnp.where(m2, s2, lv))
            nl2.append(jnp.where(m2, s2l, ll))
            nc.append(fc[a] + fc[b])
        fv1, fv2, fl1, fl2, fc = nv1, nv2, nl1, nl2, nc

    cat_v = jnp.concatenate([fv1[0], fv2[0]], axis=1)
    cat_l = jnp.concatenate([fl1[0], fl2[0]], axis=1)
    bv5, bl5 = _extract5(cat_v, cat_l, 2 * _FOLD_W)
    pad = jnp.full((nq, 3), _NEG, jnp.float32)
    bv_ref[...] = jnp.concatenate(bv5 + [pad], axis=1)
    bl_ref[...] = jnp.concatenate(bl5 + [jnp.zeros((nq, 3))], axis=1)

    collision = jnp.max(fc[0]) > 2.5

    @pl.when(collision)
    def _fallback():
        wsf = jnp.where(ws_ref[...] > tau, ws_ref[...], _NEG)
        wv5, wl5 = _extract5(wsf, labrow, eblk)
        bv_ref[...] = jnp.concatenate(wv5 + [pad], axis=1)
        bl_ref[...] = jnp.concatenate(wl5 + [jnp.zeros((nq, 3))], axis=1)

    mv = jnp.concatenate([rv_ref[...], bv_ref[...]], axis=1)
    ml = jnp.concatenate([rl_ref[...], bl_ref[...]], axis=1)
    nv5, nl5 = _extract5(mv, ml, 16)
    rv_ref[...] = jnp.concatenate(nv5 + [pad], axis=1)
    rl_ref[...] = jnp.concatenate(nl5 + [jnp.zeros((nq, 3))], axis=1)

    @pl.when(i == nblk - 1)
    def _fin():
        lab_sum = jnp.sum(rl_ref[:, :_K], axis=1)
        out_ref[0, :] = jnp.round(lab_sum / k_ref[0, 0])


def kernel(queries, embeddings, labels, k):
    nq, d = queries.shape
    n, _ = embeddings.shape
    eblk = 4096
    nblk = n // eblk

    labs3 = labels.reshape(nblk, 1, eblk)
    k_arr = jnp.asarray(k, jnp.float32).reshape(1, 1)

    out = pl.pallas_call(
        functools.partial(_knn_body, nblk, eblk),
        grid=(nblk,),
        in_specs=[
            pl.BlockSpec((nq, d), lambda i: (0, 0)),
            pl.BlockSpec((eblk, d), lambda i: (i, 0)),
            pl.BlockSpec((1, 1, eblk), lambda i: (i, 0, 0)),
            pl.BlockSpec(memory_space=pltpu.SMEM),
        ],
        out_specs=pl.BlockSpec((1, nq), lambda i: (0, 0)),
        out_shape=jax.ShapeDtypeStruct((1, nq), jnp.float32),
        scratch_shapes=[
            pltpu.VMEM((nq, d), jnp.float32),
            pltpu.VMEM((nq, 8), jnp.float32),
            pltpu.VMEM((nq, 8), jnp.float32),
            pltpu.VMEM((nq, eblk), jnp.float32),
            pltpu.VMEM((nq, 8), jnp.float32),
            pltpu.VMEM((nq, 8), jnp.float32),
        ],
        compiler_params=pltpu.CompilerParams(
            dimension_semantics=("arbitrary",),
            vmem_limit_bytes=120 * 1024 * 1024,
        ),
    )(queries, embeddings, labs3, k_arr)
    return out.reshape(nq)
